# baseline (speedup 1.0000x reference)
import jax
import jax.numpy as jnp
from jax.experimental import pallas as pl
from jax.experimental.pallas import tpu as pltpu

N = 100000
D = 128
CHUNK = 4000
NCHUNK = N // CHUNK


def _deletion_kernel(x_hbm, m_ref, v_ref, ones_ref, o_hbm, buf, in_sems, out_sems):
    for c in range(NCHUNK):
        pltpu.make_async_copy(
            x_hbm.at[pl.ds(c * CHUNK, CHUNK), :],
            buf.at[c],
            in_sems.at[c],
        ).start()

    for c in range(NCHUNK):
        pltpu.make_async_copy(
            x_hbm.at[pl.ds(c * CHUNK, CHUNK), :],
            buf.at[c],
            in_sems.at[c],
        ).wait()
        xb = buf[c]
        t = jnp.dot(xb, v_ref[...], preferred_element_type=jnp.float32)
        m_row = m_ref[c]
        mcol = jax.lax.dot_general(
            m_row, ones_ref[...],
            dimension_numbers=(((0,), (0,)), ((), ())),
            preferred_element_type=jnp.float32,
        )
        buf[c] = xb + mcol * t

    for c in range(NCHUNK):
        pltpu.make_async_copy(
            buf.at[c],
            o_hbm.at[pl.ds(c * CHUNK, CHUNK), :],
            out_sems.at[c],
        ).start()

    for c in range(NCHUNK):
        pltpu.make_async_copy(
            buf.at[c],
            o_hbm.at[pl.ds(c * CHUNK, CHUNK), :],
            out_sems.at[c],
        ).wait()


@jax.jit
def _run(x, m3d, v, ones_row):
    return pl.pallas_call(
        _deletion_kernel,
        in_specs=[
            pl.BlockSpec(memory_space=pl.ANY),
            pl.BlockSpec(memory_space=pltpu.MemorySpace.VMEM),
            pl.BlockSpec(memory_space=pltpu.MemorySpace.VMEM),
            pl.BlockSpec(memory_space=pltpu.MemorySpace.VMEM),
        ],
        out_specs=pl.BlockSpec(memory_space=pl.ANY),
        out_shape=jax.ShapeDtypeStruct((N, D), jnp.float32),
        scratch_shapes=[
            pltpu.VMEM((NCHUNK, CHUNK, D), jnp.float32),
            pltpu.SemaphoreType.DMA((NCHUNK,)),
            pltpu.SemaphoreType.DMA((NCHUNK,)),
        ],
        compiler_params=pltpu.CompilerParams(
            vmem_limit_bytes=64 * 1024 * 1024,
        ),
    )(x, m3d, v, ones_row)


def kernel(x, mask, deletion_weight):
    v = deletion_weight - jnp.eye(D, dtype=jnp.float32)
    m3d = mask.astype(jnp.float32).reshape(NCHUNK, 1, CHUNK)
    ones_row = jnp.ones((1, D), dtype=jnp.float32)
    return _run(x, m3d, v, ones_row)


# speedup vs baseline: 1.0973x; 1.0973x over previous
import jax
import jax.numpy as jnp
from jax.experimental import pallas as pl
from jax.experimental.pallas import tpu as pltpu

N = 100000
D = 128
CHUNK = 5000
NCHUNK = N // CHUNK
NBUF = 6


def _deletion_kernel(x_hbm, m_ref, v_ref, ones_ref, o_hbm,
                     xbuf, obuf, in_sems, out_sems):
    def in_copy(c, slot):
        pltpu.make_async_copy(
            x_hbm.at[pl.ds(c * CHUNK, CHUNK), :],
            xbuf.at[slot],
            in_sems.at[slot],
        ).start()

    def out_copy(c, slot):
        pltpu.make_async_copy(
            obuf.at[slot],
            o_hbm.at[pl.ds(c * CHUNK, CHUNK), :],
            out_sems.at[slot],
        ).start()

    for k in range(NBUF):
        in_copy(k, k)

    def body(c, _):
        slot = jax.lax.rem(c, NBUF)
        pltpu.make_async_copy(
            x_hbm.at[pl.ds(c * CHUNK, CHUNK), :],
            xbuf.at[slot],
            in_sems.at[slot],
        ).wait()

        @pl.when(c >= NBUF)
        def _():
            pltpu.make_async_copy(
                obuf.at[slot],
                o_hbm.at[pl.ds((c - NBUF) * CHUNK, CHUNK), :],
                out_sems.at[slot],
            ).wait()

        xb = xbuf[slot]
        t = jnp.dot(xb, v_ref[...], preferred_element_type=jnp.float32)
        m_row = m_ref[c]
        mcol = jax.lax.dot_general(
            m_row, ones_ref[...],
            dimension_numbers=(((0,), (0,)), ((), ())),
            preferred_element_type=jnp.float32,
        )
        obuf[slot] = xb + mcol * t
        out_copy(c, slot)

        @pl.when(c + NBUF < NCHUNK)
        def _():
            in_copy(c + NBUF, slot)

        return ()

    jax.lax.fori_loop(0, NCHUNK, body, (), unroll=False)

    for k in range(NBUF):
        c = NCHUNK - NBUF + k
        slot = c % NBUF
        pltpu.make_async_copy(
            obuf.at[slot],
            o_hbm.at[pl.ds(c * CHUNK, CHUNK), :],
            out_sems.at[slot],
        ).wait()


@jax.jit
def _run(x, m3d, v, ones_row):
    return pl.pallas_call(
        _deletion_kernel,
        in_specs=[
            pl.BlockSpec(memory_space=pl.ANY),
            pl.BlockSpec(memory_space=pltpu.MemorySpace.VMEM),
            pl.BlockSpec(memory_space=pltpu.MemorySpace.VMEM),
            pl.BlockSpec(memory_space=pltpu.MemorySpace.VMEM),
        ],
        out_specs=pl.BlockSpec(memory_space=pl.ANY),
        out_shape=jax.ShapeDtypeStruct((N, D), jnp.float32),
        scratch_shapes=[
            pltpu.VMEM((NBUF, CHUNK, D), jnp.float32),
            pltpu.VMEM((NBUF, CHUNK, D), jnp.float32),
            pltpu.SemaphoreType.DMA((NBUF,)),
            pltpu.SemaphoreType.DMA((NBUF,)),
        ],
    )(x, m3d, v, ones_row)


def kernel(x, mask, deletion_weight):
    v = deletion_weight - jnp.eye(D, dtype=jnp.float32)
    m3d = mask.astype(jnp.float32).reshape(NCHUNK, 1, CHUNK)
    ones_row = jnp.ones((1, D), dtype=jnp.float32)
    return _run(x, m3d, v, ones_row)


# speedup vs baseline: 1.1771x; 1.0727x over previous
import jax
import jax.numpy as jnp
from jax.experimental import pallas as pl
from jax.experimental.pallas import tpu as pltpu

N = 100000
D = 128
CHUNK = 4000
NCHUNK = N // CHUNK
NBUF_IN = 6
NBUF_OUT = 10


def _deletion_kernel(x_hbm, m_ref, v_ref, ones_ref, o_hbm,
                     xbuf, obuf, in_sems, out_sems):
    def in_copy(c, slot):
        pltpu.make_async_copy(
            x_hbm.at[pl.ds(c * CHUNK, CHUNK), :],
            xbuf.at[slot],
            in_sems.at[slot],
        ).start()

    for k in range(NBUF_IN):
        in_copy(k, k)

    def body(c, _):
        slot = jax.lax.rem(c, NBUF_IN)
        oslot = jax.lax.rem(c, NBUF_OUT)
        pltpu.make_async_copy(
            x_hbm.at[pl.ds(c * CHUNK, CHUNK), :],
            xbuf.at[slot],
            in_sems.at[slot],
        ).wait()

        @pl.when(c >= NBUF_OUT)
        def _():
            pltpu.make_async_copy(
                obuf.at[oslot],
                o_hbm.at[pl.ds((c - NBUF_OUT) * CHUNK, CHUNK), :],
                out_sems.at[oslot],
            ).wait()

        xb = xbuf[slot]
        t = jnp.dot(xb, v_ref[...], preferred_element_type=jnp.float32)
        m_row = m_ref[c]
        mcol = jax.lax.dot_general(
            m_row, ones_ref[...],
            dimension_numbers=(((0,), (0,)), ((), ())),
            preferred_element_type=jnp.float32,
        )
        obuf[oslot] = xb + mcol * t
        pltpu.make_async_copy(
            obuf.at[oslot],
            o_hbm.at[pl.ds(c * CHUNK, CHUNK), :],
            out_sems.at[oslot],
        ).start()

        @pl.when(c + NBUF_IN < NCHUNK)
        def _():
            in_copy(c + NBUF_IN, slot)

        return ()

    jax.lax.fori_loop(0, NCHUNK, body, (), unroll=False)

    for k in range(NBUF_OUT):
        c = NCHUNK - NBUF_OUT + k
        oslot = c % NBUF_OUT
        pltpu.make_async_copy(
            obuf.at[oslot],
            o_hbm.at[pl.ds(c * CHUNK, CHUNK), :],
            out_sems.at[oslot],
        ).wait()


@jax.jit
def _run(x, m3d, v, ones_row):
    return pl.pallas_call(
        _deletion_kernel,
        in_specs=[
            pl.BlockSpec(memory_space=pl.ANY),
            pl.BlockSpec(memory_space=pltpu.MemorySpace.VMEM),
            pl.BlockSpec(memory_space=pltpu.MemorySpace.VMEM),
            pl.BlockSpec(memory_space=pltpu.MemorySpace.VMEM),
        ],
        out_specs=pl.BlockSpec(memory_space=pl.ANY),
        out_shape=jax.ShapeDtypeStruct((N, D), jnp.float32),
        scratch_shapes=[
            pltpu.VMEM((NBUF_IN, CHUNK, D), jnp.float32),
            pltpu.VMEM((NBUF_OUT, CHUNK, D), jnp.float32),
            pltpu.SemaphoreType.DMA((NBUF_IN,)),
            pltpu.SemaphoreType.DMA((NBUF_OUT,)),
        ],
    )(x, m3d, v, ones_row)


def kernel(x, mask, deletion_weight):
    v = deletion_weight - jnp.eye(D, dtype=jnp.float32)
    m3d = mask.astype(jnp.float32).reshape(NCHUNK, 1, CHUNK)
    ones_row = jnp.ones((1, D), dtype=jnp.float32)
    return _run(x, m3d, v, ones_row)


# speedup vs baseline: 1.1813x; 1.0036x over previous
import jax
import jax.numpy as jnp
from jax.experimental import pallas as pl
from jax.experimental.pallas import tpu as pltpu

N = 100000
D = 128
CHUNK = 4000
NCHUNK = N // CHUNK
NBUF = 6


def _deletion_kernel(x_hbm, m_ref, v_ref, ones_ref, o_hbm,
                     xbuf, obuf, in_sems, out_sems):
    def in_copy(c, slot):
        pltpu.make_async_copy(
            x_hbm.at[pl.ds(c * CHUNK, CHUNK), :],
            xbuf.at[slot],
            in_sems.at[slot],
        ).start()

    def out_copy(c, slot):
        pltpu.make_async_copy(
            obuf.at[slot],
            o_hbm.at[pl.ds(c * CHUNK, CHUNK), :],
            out_sems.at[slot],
        ).start()

    for k in range(NBUF):
        in_copy(k, k)

    def body(c, _):
        slot = jax.lax.rem(c, NBUF)
        pltpu.make_async_copy(
            x_hbm.at[pl.ds(c * CHUNK, CHUNK), :],
            xbuf.at[slot],
            in_sems.at[slot],
        ).wait()

        @pl.when(c >= NBUF)
        def _():
            pltpu.make_async_copy(
                obuf.at[slot],
                o_hbm.at[pl.ds((c - NBUF) * CHUNK, CHUNK), :],
                out_sems.at[slot],
            ).wait()

        xb = xbuf[slot]
        t = jnp.dot(xb, v_ref[...], preferred_element_type=jnp.float32)
        m_row = m_ref[c]
        mcol = jax.lax.dot_general(
            m_row, ones_ref[...],
            dimension_numbers=(((0,), (0,)), ((), ())),
            preferred_element_type=jnp.float32,
        )
        obuf[slot] = xb + mcol * t
        out_copy(c, slot)

        @pl.when(c + NBUF < NCHUNK)
        def _():
            in_copy(c + NBUF, slot)

        return ()

    jax.lax.fori_loop(0, NCHUNK, body, (), unroll=False)

    for k in range(NBUF):
        c = NCHUNK - NBUF + k
        slot = c % NBUF
        pltpu.make_async_copy(
            obuf.at[slot],
            o_hbm.at[pl.ds(c * CHUNK, CHUNK), :],
            out_sems.at[slot],
        ).wait()


@jax.jit
def _run(x, m3d, v, ones_row):
    return pl.pallas_call(
        _deletion_kernel,
        in_specs=[
            pl.BlockSpec(memory_space=pl.ANY),
            pl.BlockSpec(memory_space=pltpu.MemorySpace.VMEM),
            pl.BlockSpec(memory_space=pltpu.MemorySpace.VMEM),
            pl.BlockSpec(memory_space=pltpu.MemorySpace.VMEM),
        ],
        out_specs=pl.BlockSpec(memory_space=pl.ANY),
        out_shape=jax.ShapeDtypeStruct((N, D), jnp.float32),
        scratch_shapes=[
            pltpu.VMEM((NBUF, CHUNK, D), jnp.float32),
            pltpu.VMEM((NBUF, CHUNK, D), jnp.float32),
            pltpu.SemaphoreType.DMA((NBUF,)),
            pltpu.SemaphoreType.DMA((NBUF,)),
        ],
    )(x, m3d, v, ones_row)


def kernel(x, mask, deletion_weight):
    v = deletion_weight - jnp.eye(D, dtype=jnp.float32)
    m3d = mask.astype(jnp.float32).reshape(NCHUNK, 1, CHUNK)
    ones_row = jnp.ones((1, D), dtype=jnp.float32)
    return _run(x, m3d, v, ones_row)
